# baseline (speedup 1.0000x reference)
import jax
import jax.numpy as jnp
from jax.experimental import pallas as pl
from jax.experimental.pallas import tpu as pltpu

_KC = 1024


def _vq_kernel(x_ref, cb_ref, out_ref, minv_ref, mini_ref, insq_ref):
    c = pl.program_id(0)
    nc = pl.num_programs(0)
    x = x_ref[...]

    @pl.when(c == 0)
    def _():
        insq_ref[...] = jnp.sum(x * x, axis=1)[None, :]

    cbc = cb_ref[...]
    cbsq = jnp.sum(cbc * cbc, axis=1, keepdims=True)
    mmT = jax.lax.dot_general(
        cbc, x * (-2.0),
        dimension_numbers=(((1,), (1,)), ((), ())),
        preferred_element_type=jnp.float32,
    )
    dist = (cbsq + insq_ref[...]) + mmT
    loc_min = jnp.min(dist, axis=0)[None, :]
    loc_idx = jnp.argmin(dist, axis=0).astype(jnp.int32)[None, :] + c * _KC

    @pl.when(c == 0)
    def _():
        minv_ref[...] = loc_min
        mini_ref[...] = loc_idx

    @pl.when(c > 0)
    def _():
        upd = loc_min < minv_ref[...]
        minv_ref[...] = jnp.where(upd, loc_min, minv_ref[...])
        mini_ref[...] = jnp.where(upd, loc_idx, mini_ref[...])

    @pl.when(c == nc - 1)
    def _():
        out_ref[...] = mini_ref[...][None, :, :]


def kernel(z_e_x, codebook):
    Bv, Nv, D = z_e_x.shape
    K = codebook.shape[0]
    M = Bv * Nv
    flat = z_e_x.reshape(M, D).astype(jnp.float32)
    cb = codebook.astype(jnp.float32)

    idx = pl.pallas_call(
        _vq_kernel,
        grid=(K // _KC,),
        in_specs=[
            pl.BlockSpec((M, D), lambda c: (0, 0)),
            pl.BlockSpec((_KC, D), lambda c: (c, 0)),
        ],
        out_specs=pl.BlockSpec((1, 1, M), lambda c: (0, 0, 0)),
        out_shape=jax.ShapeDtypeStruct((1, 1, M), jnp.int32),
        scratch_shapes=[
            pltpu.VMEM((1, M), jnp.float32),
            pltpu.VMEM((1, M), jnp.int32),
            pltpu.VMEM((1, M), jnp.float32),
        ],
    )(flat, cb)
    return idx.reshape(Bv, Nv)


# speedup vs baseline: 1.0826x; 1.0826x over previous
import jax
import jax.numpy as jnp
from jax.experimental import pallas as pl
from jax.experimental.pallas import tpu as pltpu

_KC = 2048


def _vq_kernel(x_ref, cb_ref, out_ref, minv_ref, mini_ref, insq_ref):
    c = pl.program_id(0)
    nc = pl.num_programs(0)
    x = x_ref[...]

    @pl.when(c == 0)
    def _():
        insq_ref[...] = jnp.sum(x * x, axis=1)[None, :]

    cbc = cb_ref[...]
    cbsq = jnp.sum(cbc * cbc, axis=1, keepdims=True)
    mmT = jax.lax.dot_general(
        cbc, x * (-2.0),
        dimension_numbers=(((1,), (1,)), ((), ())),
        preferred_element_type=jnp.float32,
    )
    dist = (cbsq + insq_ref[...]) + mmT
    loc_min = jnp.min(dist, axis=0)[None, :]
    loc_idx = jnp.argmin(dist, axis=0).astype(jnp.int32)[None, :] + c * _KC

    @pl.when(c == 0)
    def _():
        minv_ref[...] = loc_min
        mini_ref[...] = loc_idx

    @pl.when(c > 0)
    def _():
        upd = loc_min < minv_ref[...]
        minv_ref[...] = jnp.where(upd, loc_min, minv_ref[...])
        mini_ref[...] = jnp.where(upd, loc_idx, mini_ref[...])

    @pl.when(c == nc - 1)
    def _():
        out_ref[...] = mini_ref[...][None, :, :]


def kernel(z_e_x, codebook):
    Bv, Nv, D = z_e_x.shape
    K = codebook.shape[0]
    M = Bv * Nv
    flat = z_e_x.reshape(M, D).astype(jnp.float32)
    cb = codebook.astype(jnp.float32)

    idx = pl.pallas_call(
        _vq_kernel,
        grid=(K // _KC,),
        in_specs=[
            pl.BlockSpec((M, D), lambda c: (0, 0)),
            pl.BlockSpec((_KC, D), lambda c: (c, 0)),
        ],
        out_specs=pl.BlockSpec((1, 1, M), lambda c: (0, 0, 0)),
        out_shape=jax.ShapeDtypeStruct((1, 1, M), jnp.int32),
        scratch_shapes=[
            pltpu.VMEM((1, M), jnp.float32),
            pltpu.VMEM((1, M), jnp.int32),
            pltpu.VMEM((1, M), jnp.float32),
        ],
    )(flat, cb)
    return idx.reshape(Bv, Nv)


# speedup vs baseline: 1.1507x; 1.0629x over previous
import jax
import jax.numpy as jnp
from jax.experimental import pallas as pl
from jax.experimental.pallas import tpu as pltpu

_KC = 2048


def _vq_kernel(x_ref, cb_ref, out_ref):
    x = x_ref[...]
    xm2 = x * (-2.0)
    in_sq = jnp.sum(x * x, axis=1)[None, :]
    K = cb_ref.shape[0]

    run_min = None
    run_idx = None
    for c in range(K // _KC):
        cbc = cb_ref[c * _KC:(c + 1) * _KC, :]
        cbsq = jnp.sum(cbc * cbc, axis=1, keepdims=True)
        mmT = jax.lax.dot_general(
            cbc, xm2,
            dimension_numbers=(((1,), (1,)), ((), ())),
            preferred_element_type=jnp.float32,
        )
        dist = (cbsq + in_sq) + mmT
        loc_min = jnp.min(dist, axis=0)[None, :]
        loc_idx = jnp.argmin(dist, axis=0).astype(jnp.int32)[None, :] + (c * _KC)
        if run_min is None:
            run_min, run_idx = loc_min, loc_idx
        else:
            upd = loc_min < run_min
            run_min = jnp.where(upd, loc_min, run_min)
            run_idx = jnp.where(upd, loc_idx, run_idx)
    out_ref[...] = run_idx[None, :, :]


def kernel(z_e_x, codebook):
    Bv, Nv, D = z_e_x.shape
    K = codebook.shape[0]
    M = Bv * Nv
    flat = z_e_x.reshape(M, D).astype(jnp.float32)
    cb = codebook.astype(jnp.float32)

    idx = pl.pallas_call(
        _vq_kernel,
        grid=(1,),
        in_specs=[
            pl.BlockSpec((M, D), lambda i: (0, 0)),
            pl.BlockSpec((K, D), lambda i: (0, 0)),
        ],
        out_specs=pl.BlockSpec((1, 1, M), lambda i: (0, 0, 0)),
        out_shape=jax.ShapeDtypeStruct((1, 1, M), jnp.int32),
    )(flat, cb)
    return idx.reshape(Bv, Nv)


# speedup vs baseline: 1.2036x; 1.0459x over previous
import jax
import jax.numpy as jnp
from jax.experimental import pallas as pl
from jax.experimental.pallas import tpu as pltpu

_KC = 2048


def _vq_kernel(x_ref, cb_ref, out_ref):
    x = x_ref[...]
    xm2 = x * (-2.0)
    in_sq = jnp.sum(x * x, axis=1)[None, :]
    K = cb_ref.shape[0]

    run_min = None
    run_idx = None
    for c in range(K // _KC):
        cbc = cb_ref[c * _KC:(c + 1) * _KC, :]
        cbsq = jnp.sum(cbc * cbc, axis=1, keepdims=True)
        mmT = jax.lax.dot_general(
            cbc, xm2,
            dimension_numbers=(((1,), (1,)), ((), ())),
            preferred_element_type=jnp.float32,
        )
        dist = (cbsq + in_sq) + mmT
        loc_min = jnp.min(dist, axis=0)[None, :]
        loc_idx = jnp.argmin(dist, axis=0).astype(jnp.int32)[None, :] + (c * _KC)
        if run_min is None:
            run_min, run_idx = loc_min, loc_idx
        else:
            upd = loc_min < run_min
            run_min = jnp.where(upd, loc_min, run_min)
            run_idx = jnp.where(upd, loc_idx, run_idx)
    Nv = out_ref.shape[1]
    for b in range(out_ref.shape[0]):
        out_ref[b:b + 1, :] = run_idx[:, b * Nv:(b + 1) * Nv]


def kernel(z_e_x, codebook):
    Bv, Nv, D = z_e_x.shape
    K = codebook.shape[0]
    M = Bv * Nv
    flat = z_e_x.reshape(M, D).astype(jnp.float32)
    cb = codebook.astype(jnp.float32)

    idx = pl.pallas_call(
        _vq_kernel,
        grid=(1,),
        in_specs=[
            pl.BlockSpec((M, D), lambda i: (0, 0)),
            pl.BlockSpec((K, D), lambda i: (0, 0)),
        ],
        out_specs=pl.BlockSpec((Bv, Nv), lambda i: (0, 0)),
        out_shape=jax.ShapeDtypeStruct((Bv, Nv), jnp.int32),
    )(flat, cb)
    return idx
